# Initial kernel scaffold; baseline (speedup 1.0000x reference)
import functools

import jax
import jax.numpy as jnp
from jax import lax
from jax.experimental import pallas as pl
from jax.experimental.pallas import tpu as pltpu
from jax.experimental.pallas import tpu_sc as plsc

NC = 2
NS = 16
NW = NC * NS
CHUNK = 128


def _mesh():
    return plsc.VectorSubcoreMesh(
        core_axis_name="c", subcore_axis_name="s",
        num_cores=NC, num_subcores=NS)


@functools.lru_cache(maxsize=None)
def _make_deg_kernel(npn: int, k: int):
    rows_per_tile = npn // NS

    @functools.partial(
        pl.kernel, mesh=_mesh(),
        out_type=jax.ShapeDtypeStruct((NC, 2, npn), jnp.float32),
        scratch_types=[
            pltpu.VMEM((k, CHUNK), jnp.int32),
            pltpu.VMEM((k, CHUNK), jnp.int32),
            pltpu.VMEM((CHUNK,), jnp.float32),
            pltpu.VMEM((CHUNK,), jnp.float32),
            pltpu.VMEM_SHARED((npn,), jnp.float32),
            pltpu.VMEM_SHARED((npn,), jnp.float32),
        ],
    )
    def deg_kernel(src_hbm, dst_hbm, out_hbm,
                   src_v, dst_v, ones_v, zeros_v, dego_s, degi_s):
        cid = lax.axis_index("c")
        sid = lax.axis_index("s")
        wid = sid * NC + cid

        def fill(i, _):
            ones_v[pl.ds(i * 16, 16)] = jnp.full((16,), 1.0, jnp.float32)
            zeros_v[pl.ds(i * 16, 16)] = jnp.zeros((16,), jnp.float32)
            return 0
        lax.fori_loop(0, CHUNK // 16, fill, 0)

        base = sid * rows_per_tile

        def zrow(i, _):
            pltpu.sync_copy(zeros_v, dego_s.at[pl.ds(base + i * CHUNK, CHUNK)])
            pltpu.sync_copy(zeros_v, degi_s.at[pl.ds(base + i * CHUNK, CHUNK)])
            return 0
        lax.fori_loop(0, rows_per_tile // CHUNK, zrow, 0)
        plsc.subcore_barrier()

        pltpu.sync_copy(src_hbm.at[wid], src_v)
        pltpu.sync_copy(dst_hbm.at[wid], dst_v)

        def body(j, _):
            pltpu.sync_copy(ones_v, dego_s.at[src_v.at[j]], add=True)
            pltpu.sync_copy(ones_v, degi_s.at[dst_v.at[j]], add=True)
            return 0
        lax.fori_loop(0, k, body, 0)
        plsc.subcore_barrier()

        pltpu.sync_copy(dego_s.at[pl.ds(base, rows_per_tile)],
                        out_hbm.at[cid, 0, pl.ds(base, rows_per_tile)])
        pltpu.sync_copy(degi_s.at[pl.ds(base, rows_per_tile)],
                        out_hbm.at[cid, 1, pl.ds(base, rows_per_tile)])

    return deg_kernel


@functools.lru_cache(maxsize=None)
def _make_agg_kernel(npn: int, d: int, k: int):
    rows_per_tile = npn // NS

    @functools.partial(
        pl.kernel, mesh=_mesh(),
        out_type=jax.ShapeDtypeStruct((NC, npn, d), jnp.float32),
        scratch_types=[
            pltpu.VMEM((k, CHUNK), jnp.int32),
            pltpu.VMEM((k, CHUNK), jnp.int32),
            pltpu.VMEM((CHUNK, d), jnp.float32),
            pltpu.VMEM_SHARED((npn, d), jnp.float32),
        ],
    )
    def agg_kernel(g_hbm, src_hbm, dst_hbm, out_hbm,
                   src_v, dst_v, rows_v, agg_s):
        cid = lax.axis_index("c")
        sid = lax.axis_index("s")
        wid = sid * NC + cid

        nv = CHUNK * d // 16

        def fz(i, _):
            rows_v[i // (d // 16), pl.ds((i % (d // 16)) * 16, 16)] = (
                jnp.zeros((16,), jnp.float32))
            return 0
        lax.fori_loop(0, nv, fz, 0)

        base = sid * rows_per_tile

        def zrow(i, _):
            pltpu.sync_copy(rows_v, agg_s.at[pl.ds(base + i * CHUNK, CHUNK)])
            return 0
        lax.fori_loop(0, rows_per_tile // CHUNK, zrow, 0)
        plsc.subcore_barrier()

        pltpu.sync_copy(src_hbm.at[wid], src_v)
        pltpu.sync_copy(dst_hbm.at[wid], dst_v)

        def body(j, _):
            pltpu.sync_copy(g_hbm.at[src_v.at[j]], rows_v)
            pltpu.sync_copy(rows_v, agg_s.at[dst_v.at[j]], add=True)
            return 0
        lax.fori_loop(0, k, body, 0)
        plsc.subcore_barrier()

        pltpu.sync_copy(agg_s.at[pl.ds(base, rows_per_tile)],
                        out_hbm.at[cid, pl.ds(base, rows_per_tile)])

    return agg_kernel


def _norm_cols(degs):
    norm_out = lax.rsqrt(jnp.maximum(degs[:, 0:1] + degs[:, 1:2], 1.0))
    norm_in = lax.rsqrt(jnp.maximum(degs[:, 2:3] + degs[:, 3:4], 1.0))
    return norm_out, norm_in


def _tc_first_body(degs_ref, x_ref, w_ref, g_ref):
    norm_out, _ = _norm_cols(degs_ref[...])
    xw = jnp.dot(x_ref[...], w_ref[...], preferred_element_type=jnp.float32)
    g_ref[...] = xw * norm_out


def _tc_mid_body(degs_ref, agg_ref, b_ref, w_ref, g_ref):
    norm_out, norm_in = _norm_cols(degs_ref[...])
    agg = agg_ref[0] + agg_ref[1]
    h = jnp.maximum(agg * norm_in + b_ref[...][None, :], 0.0)
    hw = jnp.dot(h, w_ref[...], preferred_element_type=jnp.float32)
    g_ref[...] = hw * norm_out


def _tc_last_body(degs_ref, agg_ref, b_ref, out_ref):
    _, norm_in = _norm_cols(degs_ref[...])
    agg = agg_ref[0] + agg_ref[1]
    out_ref[...] = agg * norm_in + b_ref[...][None, :]


def _tc_call(body, out_shape, *args):
    return pl.pallas_call(
        body, out_shape=jax.ShapeDtypeStruct(out_shape, jnp.float32))(*args)


def kernel(features, edge_index, W1, b1, W2, b2, W3, b3):
    n, d_in = features.shape
    e = edge_index.shape[1]
    d_h = W1.shape[1]
    d_out = W3.shape[1]

    k = -(-e // (NW * CHUNK))
    ep = NW * k * CHUNK
    npn = -(-(n + 1) // (NS * CHUNK)) * (NS * CHUNK)

    src = edge_index[0]
    dst = edge_index[1]
    pad = ep - e
    src_g = jnp.pad(src, (0, pad)).reshape(NW, k, CHUNK)
    dst_s = jnp.pad(dst, (0, pad), constant_values=n).reshape(NW, k, CHUNK)
    src_d = jnp.pad(src, (0, pad), constant_values=n).reshape(NW, k, CHUNK)

    x_p = jnp.pad(features, ((0, npn - n), (0, 0)))

    deg_parts = _make_deg_kernel(npn, k)(src_d, dst_s)
    degs = jnp.transpose(deg_parts, (2, 1, 0)).reshape(npn, 4)

    agg = _make_agg_kernel(npn, d_h, k)
    g1 = _tc_call(_tc_first_body, (npn, d_h), degs, x_p, W1)
    a1 = agg(g1, src_g, dst_s)
    g2 = _tc_call(_tc_mid_body, (npn, d_h), degs, a1, b1, W2)
    a2 = agg(g2, src_g, dst_s)
    g3 = _tc_call(_tc_mid_body, (npn, d_out), degs, a2, b2, W3)
    a3 = _make_agg_kernel(npn, d_out, k)(g3, src_g, dst_s)
    logits = _tc_call(_tc_last_body, (npn, d_out), degs, a3, b3)
    return logits[:n]


# speedup vs baseline: 6.9668x; 6.9668x over previous
import functools

import jax
import jax.numpy as jnp
from jax import lax
from jax.experimental import pallas as pl
from jax.experimental.pallas import tpu as pltpu
from jax.experimental.pallas import tpu_sc as plsc

NC = 2
NS = 16
NW = NC * NS
CHUNK = 128


def _mesh():
    return plsc.VectorSubcoreMesh(
        core_axis_name="c", subcore_axis_name="s",
        num_cores=NC, num_subcores=NS)


@functools.lru_cache(maxsize=None)
def _make_deg_kernel(npn: int, k: int):
    rows_per_tile = npn // NS

    @functools.partial(
        pl.kernel, mesh=_mesh(),
        out_type=jax.ShapeDtypeStruct((NC, 2, npn), jnp.float32),
        scratch_types=[
            pltpu.VMEM((k, CHUNK), jnp.int32),
            pltpu.VMEM((k, CHUNK), jnp.int32),
            pltpu.VMEM((CHUNK,), jnp.float32),
            pltpu.VMEM((CHUNK,), jnp.float32),
            pltpu.VMEM_SHARED((npn,), jnp.float32),
            pltpu.VMEM_SHARED((npn,), jnp.float32),
        ],
    )
    def deg_kernel(src_hbm, dst_hbm, out_hbm,
                   src_v, dst_v, ones_v, zeros_v, dego_s, degi_s):
        cid = lax.axis_index("c")
        sid = lax.axis_index("s")
        wid = sid * NC + cid

        def fill(i, _):
            ones_v[pl.ds(i * 16, 16)] = jnp.full((16,), 1.0, jnp.float32)
            zeros_v[pl.ds(i * 16, 16)] = jnp.zeros((16,), jnp.float32)
            return 0
        lax.fori_loop(0, CHUNK // 16, fill, 0)

        base = sid * rows_per_tile

        def zrow(i, _):
            pltpu.sync_copy(zeros_v, dego_s.at[pl.ds(base + i * CHUNK, CHUNK)])
            pltpu.sync_copy(zeros_v, degi_s.at[pl.ds(base + i * CHUNK, CHUNK)])
            return 0
        lax.fori_loop(0, rows_per_tile // CHUNK, zrow, 0)
        plsc.subcore_barrier()

        pltpu.sync_copy(src_hbm.at[wid], src_v)
        pltpu.sync_copy(dst_hbm.at[wid], dst_v)

        def body(j, _):
            pltpu.sync_copy(ones_v, dego_s.at[src_v.at[j]], add=True)
            pltpu.sync_copy(ones_v, degi_s.at[dst_v.at[j]], add=True)
            return 0
        lax.fori_loop(0, k, body, 0)
        plsc.subcore_barrier()

        pltpu.sync_copy(dego_s.at[pl.ds(base, rows_per_tile)],
                        out_hbm.at[cid, 0, pl.ds(base, rows_per_tile)])
        pltpu.sync_copy(degi_s.at[pl.ds(base, rows_per_tile)],
                        out_hbm.at[cid, 1, pl.ds(base, rows_per_tile)])

    return deg_kernel


@functools.lru_cache(maxsize=None)
def _make_agg_kernel(npn: int, d: int, k: int):
    rows_per_tile = npn // NS

    @functools.partial(
        pl.kernel, mesh=_mesh(),
        out_type=jax.ShapeDtypeStruct((NC, npn, d), jnp.float32),
        scratch_types=[
            pltpu.VMEM((k, CHUNK), jnp.int32),
            pltpu.VMEM((k, CHUNK), jnp.int32),
            pltpu.VMEM((CHUNK, d), jnp.float32),
            pltpu.VMEM_SHARED((npn, d), jnp.float32),
        ],
    )
    def agg_kernel(g_hbm, src_hbm, dst_hbm, out_hbm,
                   src_v, dst_v, rows_v, agg_s):
        cid = lax.axis_index("c")
        sid = lax.axis_index("s")
        wid = sid * NC + cid

        nv = CHUNK * d // 16

        def fz(i, _):
            rows_v[i // (d // 16), pl.ds((i % (d // 16)) * 16, 16)] = (
                jnp.zeros((16,), jnp.float32))
            return 0
        lax.fori_loop(0, nv, fz, 0)

        base = sid * rows_per_tile

        def zrow(i, _):
            pltpu.sync_copy(rows_v, agg_s.at[pl.ds(base + i * CHUNK, CHUNK)])
            return 0
        lax.fori_loop(0, rows_per_tile // CHUNK, zrow, 0)
        plsc.subcore_barrier()

        pltpu.sync_copy(src_hbm.at[wid], src_v)
        pltpu.sync_copy(dst_hbm.at[wid], dst_v)

        def body(j, _):
            pltpu.sync_copy(g_hbm.at[src_v.at[j]], rows_v)
            pltpu.sync_copy(rows_v, agg_s.at[dst_v.at[j]], add=True)
            return 0
        lax.fori_loop(0, k, body, 0)
        plsc.subcore_barrier()

        pltpu.sync_copy(agg_s.at[pl.ds(base, rows_per_tile)],
                        out_hbm.at[cid, pl.ds(base, rows_per_tile)])

    return agg_kernel


def _norm_cols(degs):
    norm_out = lax.rsqrt(jnp.maximum(degs[:, 0:1] + degs[:, 1:2], 1.0))
    norm_in = lax.rsqrt(jnp.maximum(degs[:, 2:3] + degs[:, 3:4], 1.0))
    return norm_out, norm_in


def _tc_first_body(degs_ref, x_ref, w_ref, g_ref):
    norm_out, _ = _norm_cols(degs_ref[...])
    xw = jnp.dot(x_ref[...], w_ref[...], preferred_element_type=jnp.float32)
    g_ref[...] = xw * norm_out


def _tc_mid_body(degs_ref, agg_ref, b_ref, w_ref, g_ref):
    norm_out, norm_in = _norm_cols(degs_ref[...])
    agg = agg_ref[0] + agg_ref[1]
    h = jnp.maximum(agg * norm_in + b_ref[...][None, :], 0.0)
    hw = jnp.dot(h, w_ref[...], preferred_element_type=jnp.float32)
    g_ref[...] = hw * norm_out


def _tc_last_body(degs_ref, agg_ref, b_ref, out_ref):
    _, norm_in = _norm_cols(degs_ref[...])
    agg = agg_ref[0] + agg_ref[1]
    out_ref[...] = agg * norm_in + b_ref[...][None, :]


def _tc_call(body, out_shape, *args):
    return pl.pallas_call(
        body, out_shape=jax.ShapeDtypeStruct(out_shape, jnp.float32))(*args)


def kernel(features, edge_index, W1, b1, W2, b2, W3, b3):
    n, d_in = features.shape
    e = edge_index.shape[1]
    d_h = W1.shape[1]
    d_out = W3.shape[1]

    k = -(-e // (NW * CHUNK))
    ep = NW * k * CHUNK
    npn = -(-(n + 1) // (NS * CHUNK)) * (NS * CHUNK)

    src = edge_index[0]
    dst = edge_index[1]
    pad = ep - e
    src_g = jnp.pad(src, (0, pad)).reshape(NW, k, CHUNK)
    dst_s = jnp.pad(dst, (0, pad), constant_values=n).reshape(NW, k, CHUNK)
    src_d = jnp.pad(src, (0, pad), constant_values=n).reshape(NW, k, CHUNK)

    x_p = jnp.pad(features, ((0, npn - n), (0, 0)))

    deg_parts = _make_deg_kernel(npn, k)(src_d, dst_s)
    degs = jnp.transpose(deg_parts, (2, 1, 0)).reshape(npn, 4)

    w3_p = jnp.pad(W3, ((0, 0), (0, d_h - d_out)))
    b3_p = jnp.pad(b3, (0, d_h - d_out))

    agg = _make_agg_kernel(npn, d_h, k)
    g1 = _tc_call(_tc_first_body, (npn, d_h), degs, x_p, W1)
    a1 = agg(g1, src_g, dst_s)
    g2 = _tc_call(_tc_mid_body, (npn, d_h), degs, a1, b1, W2)
    a2 = agg(g2, src_g, dst_s)
    g3 = _tc_call(_tc_mid_body, (npn, d_h), degs, a2, b2, w3_p)
    a3 = agg(g3, src_g, dst_s)
    logits = _tc_call(_tc_last_body, (npn, d_h), degs, a3, b3_p)
    return logits[:n, :d_out]
